# baseline (speedup 1.0000x reference)
import functools

import jax
import jax.numpy as jnp
from jax import lax
from jax.experimental import pallas as pl
from jax.experimental.pallas import tpu as pltpu
from jax.experimental.pallas import tpu_sc as plsc

VOCAB_SIZE = 100000
EMB_D = 16
B = 1024


def _build_sc_gather():
    info = plsc.get_sparse_core_info()
    nc, ns = info.num_cores, info.num_subcores
    nw = nc * ns
    w_per_w = B * EMB_D // nw
    chunk = 128
    nchunks = w_per_w // chunk

    mesh = plsc.VectorSubcoreMesh(core_axis_name="c", subcore_axis_name="s")

    @functools.partial(
        pl.kernel,
        mesh=mesh,
        out_type=jax.ShapeDtypeStruct((B * EMB_D,), jnp.float32),
        scratch_types=[
            pltpu.VMEM((w_per_w,), jnp.int32),
            pltpu.VMEM((w_per_w,), jnp.float32),
            pltpu.SemaphoreType.DMA,
        ],
    )
    def gather_kernel(idx_hbm, table_hbm, out_hbm, idx_v, rows_v, sem):
        wid = lax.axis_index("s") * nc + lax.axis_index("c")
        base = wid * w_per_w
        pltpu.sync_copy(idx_hbm.at[pl.ds(base, w_per_w)], idx_v)
        copies = [
            pltpu.async_copy(
                table_hbm.at[idx_v.at[pl.ds(s * chunk, chunk)]],
                rows_v.at[pl.ds(s * chunk, chunk)],
                sem,
            )
            for s in range(nchunks)
        ]
        for c in copies:
            c.wait()
        pltpu.sync_copy(rows_v, out_hbm.at[pl.ds(base, w_per_w)])

    return gather_kernel


_sc_gather = _build_sc_gather()


TILE_V = 2048


def _matmul_body(wt_ref, embt_ref, b_ref, out_ref):
    acc = lax.dot_general(
        wt_ref[...],
        embt_ref[...],
        dimension_numbers=(((0,), (0,)), ((), ())),
        preferred_element_type=jnp.float32,
    )
    out_ref[...] = acc + b_ref[...]


def _tc_matmul_t(fc_wt, embedded, fc_b2d):
    return pl.pallas_call(
        _matmul_body,
        grid=((VOCAB_SIZE + TILE_V - 1) // TILE_V,),
        in_specs=[
            pl.BlockSpec((EMB_D, TILE_V), lambda v: (0, v)),
            pl.BlockSpec((EMB_D, B), lambda v: (0, 0)),
            pl.BlockSpec((TILE_V, 1), lambda v: (v, 0)),
        ],
        out_specs=pl.BlockSpec((TILE_V, B), lambda v: (v, 0)),
        out_shape=jax.ShapeDtypeStruct((VOCAB_SIZE, B), jnp.float32),
    )(fc_wt, embedded, fc_b2d)


def kernel(x, emb_table, fc_w, fc_b):
    xi = x.astype(jnp.int32)
    word_idx = (
        jnp.arange(EMB_D, dtype=jnp.int32)[:, None] + xi[None, :] * EMB_D
    ).reshape(-1)
    embedded_flat = _sc_gather(word_idx, emb_table.reshape(-1))
    embedded_t = embedded_flat.reshape(EMB_D, B)
    logits_t = _tc_matmul_t(fc_w.T, embedded_t, fc_b.reshape(VOCAB_SIZE, 1))
    return logits_t.T


# speedup vs baseline: 1.1808x; 1.1808x over previous
import functools

import jax
import jax.numpy as jnp
from jax import lax
from jax.experimental import pallas as pl
from jax.experimental.pallas import tpu as pltpu
from jax.experimental.pallas import tpu_sc as plsc

VOCAB_SIZE = 100000
EMB_D = 16
B = 1024


def _build_sc_gather():
    info = plsc.get_sparse_core_info()
    nc, ns = info.num_cores, info.num_subcores
    nw = nc * ns
    w_per_w = B * EMB_D // nw
    chunk = 128
    nchunks = w_per_w // chunk

    mesh = plsc.VectorSubcoreMesh(core_axis_name="c", subcore_axis_name="s")

    @functools.partial(
        pl.kernel,
        mesh=mesh,
        out_type=jax.ShapeDtypeStruct((B * EMB_D,), jnp.float32),
        scratch_types=[
            pltpu.VMEM((w_per_w,), jnp.int32),
            pltpu.VMEM((w_per_w,), jnp.float32),
            pltpu.SemaphoreType.DMA,
        ],
    )
    def gather_kernel(idx_hbm, table_hbm, out_hbm, idx_v, rows_v, sem):
        wid = lax.axis_index("s") * nc + lax.axis_index("c")
        base = wid * w_per_w
        pltpu.sync_copy(idx_hbm.at[pl.ds(base, w_per_w)], idx_v)
        copies = [
            pltpu.async_copy(
                table_hbm.at[idx_v.at[pl.ds(s * chunk, chunk)]],
                rows_v.at[pl.ds(s * chunk, chunk)],
                sem,
            )
            for s in range(nchunks)
        ]
        for c in copies:
            c.wait()
        pltpu.sync_copy(rows_v, out_hbm.at[pl.ds(base, w_per_w)])

    return gather_kernel


_sc_gather = _build_sc_gather()


TILE_V = 2048


def _matmul_body(wt_ref, embt_ref, b_ref, out_ref):
    acc = lax.dot_general(
        wt_ref[...],
        embt_ref[...],
        dimension_numbers=(((0,), (0,)), ((), ())),
        preferred_element_type=jnp.float32,
    )
    out_ref[...] = acc + b_ref[...]


def _tc_matmul_t(fc_wt, embedded, fc_b2d):
    return pl.pallas_call(
        _matmul_body,
        grid=((VOCAB_SIZE + TILE_V - 1) // TILE_V,),
        in_specs=[
            pl.BlockSpec((EMB_D, TILE_V), lambda v: (0, v)),
            pl.BlockSpec((EMB_D, B), lambda v: (0, 0)),
            pl.BlockSpec((TILE_V, 1), lambda v: (v, 0)),
        ],
        out_specs=pl.BlockSpec((TILE_V, B), lambda v: (v, 0)),
        out_shape=jax.ShapeDtypeStruct((VOCAB_SIZE, B), jnp.float32),
    )(fc_wt, embedded, fc_b2d)


def kernel(x, emb_table, fc_w, fc_b):
    xi = x.astype(jnp.int32)
    word_idx = (
        jnp.arange(EMB_D, dtype=jnp.int32)[:, None] * VOCAB_SIZE + xi[None, :]
    ).reshape(-1)
    embedded_flat = _sc_gather(word_idx, emb_table.T.reshape(-1))
    embedded_t = embedded_flat.reshape(EMB_D, B)
    logits_t = _tc_matmul_t(fc_w.T, embedded_t, fc_b.reshape(VOCAB_SIZE, 1))
    return logits_t.T
